# baseline (speedup 1.0000x reference)
import functools

import jax
import jax.numpy as jnp
from jax import lax
from jax.experimental import pallas as pl
from jax.experimental.pallas import tpu as pltpu
from jax.experimental.pallas import tpu_sc as plsc

SEQ = 200
D = 64
DP = 128
NUM_CORES = 2
NUM_SUBCORES = 16
NUM_WORKERS = NUM_CORES * NUM_SUBCORES
LANES = 16
BB = 128
LC = 8
NCHUNK = SEQ // LC


def kernel(inputs, token_table, position_table):
    batch, seq = inputs.shape
    vocab, d = token_table.shape
    mblocks = batch * d // DP

    idx_t = inputs.astype(jnp.int32).T
    tab_p = jnp.pad(token_table, ((0, 0), (0, DP - d)))

    mesh = plsc.VectorSubcoreMesh(core_axis_name="c", subcore_axis_name="s")

    @functools.partial(
        pl.kernel,
        out_type=jax.ShapeDtypeStruct((seq, mblocks, DP), jnp.float32),
        mesh=mesh,
        scratch_types=[
            pltpu.VMEM((LC, BB), jnp.int32),
            pltpu.VMEM((SEQ, DP), jnp.float32),
            pltpu.VMEM((BB, DP), jnp.float32),
            pltpu.VMEM((BB, DP), jnp.float32),
            pltpu.VMEM((8, LC, 8, DP), jnp.float32),
            pltpu.SemaphoreType.DMA,
            pltpu.SemaphoreType.DMA,
            pltpu.SemaphoreType.DMA,
        ],
        compiler_params=pltpu.CompilerParams(
            use_tc_tiling_on_sc=True, needs_layout_passes=False
        ),
    )
    def sc_embed(idx_hbm, tab_hbm, pos_hbm, out_hbm, idxc, pos_v, rowsa,
                 rowsb, tr, sem_ga, sem_gb, sem_o):
        wid = lax.axis_index("s") * NUM_CORES + lax.axis_index("c")
        rows = (rowsa, rowsb)
        sem_g = (sem_ga, sem_gb)
        rj = [jnp.arange(16, dtype=jnp.int32) + 16 * jb for jb in range(8)]

        def issue_gather(li, p):
            pltpu.async_copy(tab_hbm.at[idxc.at[li]], rows[p], sem_g[p])

        def wait_gather(li, p):
            pltpu.make_async_copy(
                tab_hbm.at[idxc.at[li]], rows[p], sem_g[p]
            ).wait()

        def out_dst(l0, s):
            return out_hbm.at[pl.ds(l0, LC), pl.ds(s * 256 + wid * 8, 8), :]

        def transpose_one(li, l, p):
            l_spl = lax.broadcast_in_dim(l, (16,), ())

            @pl.loop(0, 8)
            def _(s):
                for i in range(8):
                    dd = 8 * s + i
                    d_spl = lax.broadcast_in_dim(dd, (16,), ())
                    pos_spl = plsc.load_gather(pos_v, [l_spl, d_spl])
                    for jb in range(8):
                        v = plsc.load_gather(rows[p], [rj[jb], d_spl])
                        tr.at[s, li, i, pl.ds(16 * jb, LANES)][...] = (
                            v + pos_spl
                        )

        pltpu.sync_copy(pos_hbm, pos_v)

        @pl.loop(0, NCHUNK)
        def _(t):
            l0 = t * LC
            pltpu.sync_copy(
                idx_hbm.at[pl.ds(l0, LC), pl.ds(wid * BB, BB)], idxc
            )
            issue_gather(0, 0)

            @pl.when(t > 0)
            def _():
                for s in range(8):
                    pltpu.make_async_copy(
                        tr.at[s], out_dst(l0 - LC, s), sem_o
                    ).wait()

            @pl.loop(0, LC // 2)
            def _(lp):
                li0 = 2 * lp
                wait_gather(li0, 0)
                issue_gather(li0 + 1, 1)
                transpose_one(li0, l0 + li0, 0)

                li1 = 2 * lp + 1
                wait_gather(li1, 1)

                @pl.when(li1 + 1 < LC)
                def _():
                    issue_gather(li1 + 1, 0)

                transpose_one(li1, l0 + li1, 1)

            for s in range(8):
                pltpu.async_copy(tr.at[s], out_dst(l0, s), sem_o)

        for s in range(8):
            pltpu.make_async_copy(
                tr.at[s], out_dst(SEQ - LC, s), sem_o
            ).wait()

    pos_p = jnp.pad(position_table, ((0, 0), (0, DP - d)))
    out = sc_embed(idx_t, tab_p, pos_p)
    out5 = out.reshape(seq, 8, batch // DP, 8, DP)
    return out5.transpose(2, 4, 0, 1, 3).reshape(batch, seq, d)


# speedup vs baseline: 2.7392x; 2.7392x over previous
import functools

import jax
import jax.numpy as jnp
from jax import lax
from jax.experimental import pallas as pl
from jax.experimental.pallas import tpu as pltpu
from jax.experimental.pallas import tpu_sc as plsc

SEQ = 200
D = 64
DP = 128
NUM_CORES = 2
NUM_SUBCORES = 16
NUM_WORKERS = NUM_CORES * NUM_SUBCORES
LANES = 16
NBUF = 3
G0, G1 = 128, 72


def kernel(inputs, token_table, position_table):
    batch, seq = inputs.shape
    vocab, d = token_table.shape
    total = batch * seq
    rows_per_w = total // NUM_WORKERS
    seq_per_w = rows_per_w // seq

    idx_flat = inputs.reshape(total).astype(jnp.int32)
    tab_p = jnp.pad(token_table, ((0, 0), (0, DP - d)))
    pos_p = jnp.pad(position_table, ((0, 0), (0, DP - d)))

    mesh = plsc.VectorSubcoreMesh(core_axis_name="c", subcore_axis_name="s")

    @functools.partial(
        pl.kernel,
        out_type=jax.ShapeDtypeStruct((batch, seq, DP), jnp.float32),
        mesh=mesh,
        scratch_types=[
            pltpu.VMEM((rows_per_w,), jnp.int32),
            pltpu.VMEM((SEQ, DP), jnp.float32),
            pltpu.VMEM((SEQ, DP), jnp.float32),
            pltpu.VMEM((SEQ, DP), jnp.float32),
            pltpu.VMEM((SEQ, DP), jnp.float32),
            pltpu.SemaphoreType.DMA,
            pltpu.SemaphoreType.DMA,
            pltpu.SemaphoreType.DMA,
            pltpu.SemaphoreType.DMA,
            pltpu.SemaphoreType.DMA,
            pltpu.SemaphoreType.DMA,
        ],
        compiler_params=pltpu.CompilerParams(use_tc_tiling_on_sc=True),
    )
    def sc_embed(idx_hbm, tab_hbm, pos_hbm, out_hbm, idx_v, pos_v, rows0,
                 rows1, rows2, g0, g1, g2, o0, o1, o2):
        wid = lax.axis_index("s") * NUM_CORES + lax.axis_index("c")
        base = wid * rows_per_w
        rows = (rows0, rows1, rows2)
        sem_g = (g0, g1, g2)
        sem_o = (o0, o1, o2)

        def issue_gather(s, b):
            o = s * SEQ
            pltpu.async_copy(
                tab_hbm.at[idx_v.at[pl.ds(o, G0)]], rows[b].at[pl.ds(0, G0)],
                sem_g[b],
            )
            pltpu.async_copy(
                tab_hbm.at[idx_v.at[pl.ds(o + G0, G1)]],
                rows[b].at[pl.ds(G0, G1)], sem_g[b],
            )

        def wait_gather(s, b):
            o = s * SEQ
            pltpu.make_async_copy(
                tab_hbm.at[idx_v.at[pl.ds(o, G0)]], rows[b].at[pl.ds(0, G0)],
                sem_g[b],
            ).wait()
            pltpu.make_async_copy(
                tab_hbm.at[idx_v.at[pl.ds(o + G0, G1)]],
                rows[b].at[pl.ds(G0, G1)], sem_g[b],
            ).wait()

        def issue_out(s, b):
            pltpu.async_copy(rows[b], out_hbm.at[wid * seq_per_w + s],
                             sem_o[b])

        def wait_out(s, b):
            pltpu.make_async_copy(
                rows[b], out_hbm.at[wid * seq_per_w + s], sem_o[b]
            ).wait()

        def add_pos(b):
            @pl.loop(0, SEQ)
            def _(r):
                for j in range(D // LANES):
                    sl = (pl.ds(r, 1), pl.ds(j * LANES, LANES))
                    plsc.addupdate(rows[b].at[sl], pos_v.at[sl][...])

        pltpu.sync_copy(idx_hbm.at[pl.ds(base, rows_per_w)], idx_v)
        pltpu.sync_copy(pos_hbm, pos_v)

        issue_gather(0, 0)
        issue_gather(1, 1)
        wait_gather(0, 0)
        add_pos(0)
        issue_out(0, 0)
        issue_gather(2, 2)
        wait_gather(1, 1)
        add_pos(1)
        issue_out(1, 1)
        wait_out(0, 0)
        issue_gather(3, 0)

        @pl.loop(0, (seq_per_w - 2) // NBUF)
        def _(i):
            for k in range(NBUF):
                s = NBUF * i + 2 + k
                b = (2 + k) % NBUF
                z = (1 + k) % NBUF
                wait_gather(s, b)
                add_pos(b)
                issue_out(s, b)
                wait_out(s - 1, z)

                @pl.when(s + 2 < seq_per_w)
                def _():
                    issue_gather(s + 2, z)

        wait_out(seq_per_w - 1, (seq_per_w - 1) % NBUF)

    out = sc_embed(idx_flat, tab_p, pos_p)
    return out[:, :, :D]
